# baseline (speedup 1.0000x reference)
import functools

import jax
import jax.numpy as jnp
import numpy as np
from jax import lax
from jax.experimental import pallas as pl
from jax.experimental.pallas import tpu as pltpu
from jax.experimental.pallas import tpu_sc as plsc

B = 16384
L = 50
D = 32

NC = 2
NS = 16
NW = NC * NS
BPW = B // NW

V = 1000000
BWW = 768
NBLK = (V // 128 * 128) // BWW
TAIL = V - NBLK * BWW
OPB = BWW * D // 128
OROWS = V * D // 128
NBW = 42
NPAIR = NBW // 2

FCH = 32
NFCH = BPW // FCH

CH = 64
NCHUNK = BPW // CH
GPC = CH * L // 128

ITCH = 128
NIT = BPW // ITCH

_mesh = plsc.VectorSubcoreMesh(
    core_axis_name="c", subcore_axis_name="s", num_cores=NC, num_subcores=NS
)


def _transpose_block(v2, vt):
    iota32 = lax.iota(jnp.int32, 16) * D

    @plsc.parallel_loop(0, BWW // 16, unroll=4)
    def group(g):
        w0 = g * 16
        gbase = w0 * D
        for f in range(D):
            vals = v2[f, pl.ds(w0, 16)]
            plsc.store_scatter(vt, [iota32 + (gbase + f)], vals)


def _convert_table(tabT_hbm, out_hbm, wid, v2a, v2b, vta, vtb,
                   isemA, isemB, osemA, osemB):
    def blkno(k):
        return jnp.minimum(k * NW + wid, NBLK - 1)

    def src_at(k):
        return tabT_hbm.at[:, pl.ds(blkno(k) * BWW, BWW)]

    def dst_at(k):
        return out_hbm.at[pl.ds(blkno(k) * (BWW * D), BWW * D)]

    pltpu.async_copy(src_at(0), v2a, isemA)
    pltpu.async_copy(src_at(1), v2b, isemB)

    def pair(q, carry):
        k = q * 2
        pltpu.make_async_copy(src_at(0), v2a, isemA).wait()
        _transpose_block(v2a, vta)
        outA = pltpu.async_copy(vta, dst_at(k), osemA)
        pltpu.async_copy(src_at_dyn_a(k + 2), v2a, isemA)
        pltpu.make_async_copy(src_at(1), v2b, isemB).wait()
        _transpose_block(v2b, vtb)
        outB = pltpu.async_copy(vtb, dst_at(k + 1), osemB)
        pltpu.async_copy(src_at_dyn_b(k + 3), v2b, isemB)
        outA.wait()
        outB.wait()
        return carry

    def src_at_dyn_a(k):
        return src_at(k)

    def src_at_dyn_b(k):
        return src_at(k)

    lax.fori_loop(0, NPAIR, pair, 0)
    pltpu.make_async_copy(src_at(0), v2a, isemA).wait()
    pltpu.make_async_copy(src_at(1), v2b, isemB).wait()


def _convert_tail(tail_hbm, out_hbm, v2t, vtt):
    pltpu.sync_copy(tail_hbm, v2t)

    def row(w, carry):
        vtt[pl.ds(w * D, 16)] = v2t[w, pl.ds(0, 16)]
        vtt[pl.ds(w * D + 16, 16)] = v2t[w, pl.ds(16, 16)]
        return carry

    lax.fori_loop(0, TAIL, row, 0)
    pltpu.sync_copy(vtt, out_hbm.at[pl.ds((V - TAIL) * D, TAIL * D)])


@functools.partial(
    pl.kernel,
    out_type=(
        jax.ShapeDtypeStruct((V * D,), jnp.float32),
        jax.ShapeDtypeStruct((V * D,), jnp.float32),
        jax.ShapeDtypeStruct((B * L,), jnp.int32),
    ),
    mesh=_mesh,
    compiler_params=pltpu.CompilerParams(
        use_tc_tiling_on_sc=True, needs_layout_passes=False
    ),
    scratch_types=[
        pltpu.VMEM((32, BWW), jnp.float32),
        pltpu.VMEM((32, BWW), jnp.float32),
        pltpu.VMEM((BWW * D,), jnp.float32),
        pltpu.VMEM((BWW * D,), jnp.float32),
        pltpu.VMEM((TAIL, D), jnp.float32),
        pltpu.VMEM((TAIL * D,), jnp.float32),
        pltpu.VMEM((FCH, L), jnp.int32),
        pltpu.VMEM((FCH * L,), jnp.int32),
        pltpu.SemaphoreType.DMA,
        pltpu.SemaphoreType.DMA,
        pltpu.SemaphoreType.DMA,
        pltpu.SemaphoreType.DMA,
    ],
)
def _sc_format(
    wordT_hbm, itemT_hbm, wtail_hbm, itail_hbm, qw_hbm,
    wout_hbm, iout_hbm, qwf_hbm,
    v2a, v2b, vta, vtb, v2t, vtt, vf2, vf1,
    isemA, isemB, osemA, osemB,
):
    wid = lax.axis_index("s") * NC + lax.axis_index("c")
    _convert_table(wordT_hbm, wout_hbm, wid, v2a, v2b, vta, vtb,
                   isemA, isemB, osemA, osemB)
    _convert_table(itemT_hbm, iout_hbm, wid, v2a, v2b, vta, vtb,
                   isemA, isemB, osemA, osemB)
    _convert_tail(wtail_hbm, wout_hbm, v2t, vtt)
    _convert_tail(itail_hbm, iout_hbm, v2t, vtt)

    base = wid * BPW

    def fchunk(c, carry):
        pltpu.sync_copy(qw_hbm.at[pl.ds(base + c * FCH, FCH)], vf2)

        def row(r, rc):
            o = r * L
            vf1[pl.ds(o, 16)] = vf2[r, pl.ds(0, 16)]
            vf1[pl.ds(o + 16, 16)] = vf2[r, pl.ds(16, 16)]
            vf1[pl.ds(o + 32, 16)] = vf2[r, pl.ds(32, 16)]
            vf1[pl.ds(o + 34, 16)] = vf2[r, pl.ds(34, 16)]
            return rc

        lax.fori_loop(0, FCH, row, 0)
        pltpu.sync_copy(vf1, qwf_hbm.at[pl.ds((base + c * FCH) * L, FCH * L)])
        return carry

    lax.fori_loop(0, NFCH, fchunk, 0)


@functools.partial(
    pl.kernel,
    out_type=(
        jax.ShapeDtypeStruct((B, D), jnp.float32),
        jax.ShapeDtypeStruct((B, D), jnp.float32),
    ),
    mesh=_mesh,
    compiler_params=pltpu.CompilerParams(use_tc_tiling_on_sc=False),
    scratch_types=[
        pltpu.VMEM((CH * L,), jnp.int32),
        pltpu.VMEM((CH * L, D), jnp.float32),
        pltpu.VMEM((CH, D), jnp.float32),
        pltpu.VMEM((BPW,), jnp.int32),
        pltpu.VMEM((ITCH, D), jnp.float32),
        pltpu.SemaphoreType.DMA,
    ],
)
def _sc_gather_pool(
    items_hbm, qwf_hbm, word_hbm, item_hbm,
    pooled_hbm, iout_hbm,
    idx_v, rows_v, pooled_v, iidx_v, irows_v, sem,
):
    wid = lax.axis_index("s") * NC + lax.axis_index("c")
    base = wid * BPW

    pltpu.sync_copy(items_hbm.at[pl.ds(base, BPW)], iidx_v)

    def chunk_body(c, carry):
        pltpu.sync_copy(
            qwf_hbm.at[pl.ds((base + c * CH) * L, CH * L)], idx_v
        )
        cps = [
            pltpu.async_copy(
                word_hbm.at[idx_v.at[pl.ds(j * 128, 128)]],
                rows_v.at[pl.ds(j * 128, 128)],
                sem,
            )
            for j in range(GPC)
        ]
        for cp in cps:
            cp.wait()

        def item_body(b, acc_carry):
            r0 = b * L
            acc0 = rows_v[r0, pl.ds(0, 16)]
            acc1 = rows_v[r0, pl.ds(16, 16)]
            for l in range(1, L):
                acc0 = acc0 + rows_v[r0 + l, pl.ds(0, 16)]
                acc1 = acc1 + rows_v[r0 + l, pl.ds(16, 16)]
            pooled_v[b, pl.ds(0, 16)] = acc0
            pooled_v[b, pl.ds(16, 16)] = acc1
            return acc_carry

        lax.fori_loop(0, CH, item_body, 0)
        pltpu.sync_copy(pooled_v, pooled_hbm.at[pl.ds(base + c * CH, CH)])
        return carry

    lax.fori_loop(0, NCHUNK, chunk_body, 0)

    for t in range(NIT):
        pltpu.async_copy(
            item_hbm.at[iidx_v.at[pl.ds(t * ITCH, ITCH)]], irows_v, sem
        ).wait()
        pltpu.sync_copy(irows_v, iout_hbm.at[pl.ds(base + t * ITCH, ITCH)])


_TB = 2048


def _tc_proj_body(x_ref, w_ref, b_ref, o_ref):
    x = x_ref[...] * np.float32(1.0 / L)
    y = lax.dot_general(
        x, w_ref[...], (((1,), (1,)), ((), ())),
        preferred_element_type=jnp.float32,
    )
    o_ref[...] = jnp.tanh(y + b_ref[...])


_tc_proj = pl.pallas_call(
    _tc_proj_body,
    out_shape=jax.ShapeDtypeStruct((B, D), jnp.float32),
    grid=(B // _TB,),
    in_specs=[
        pl.BlockSpec((_TB, D), lambda i: (i, 0)),
        pl.BlockSpec((D, D), lambda i: (0, 0)),
        pl.BlockSpec((1, D), lambda i: (0, 0)),
    ],
    out_specs=pl.BlockSpec((_TB, D), lambda i: (i, 0)),
)


def kernel(items, query_words, word_table, item_table, W_q, b_q):
    items = items.astype(jnp.int32)
    query_words = query_words.astype(jnp.int32)
    word_c, item_c, qw_flat = _sc_format(
        word_table.T, item_table.T,
        word_table[NBLK * BWW:], item_table[NBLK * BWW:],
        query_words,
    )
    word_2 = word_c.reshape(V, D)
    item_2 = item_c.reshape(V, D)
    pooled_sum, item_emb = _sc_gather_pool(
        items, qw_flat, word_2, item_2
    )
    q = _tc_proj(pooled_sum, W_q, b_q.reshape(1, D))
    return (q, item_emb)


# speedup vs baseline: 1.4627x; 1.4627x over previous
import functools

import jax
import jax.numpy as jnp
import numpy as np
from jax import lax
from jax.experimental import pallas as pl
from jax.experimental.pallas import tpu as pltpu
from jax.experimental.pallas import tpu_sc as plsc

B = 16384
L = 50
D = 32

NC = 2
NS = 16
NW = NC * NS
BPW = B // NW

V = 1000000
BWW = 768
NBLK = (V // 128 * 128) // BWW
TAIL = V - NBLK * BWW
OPB = BWW * D // 128
OROWS = V * D // 128
NBW = 42
NPAIR = NBW // 2

FCH = 32
NFCH = BPW // FCH

CH = 64
NCHUNK = BPW // CH
GPC = CH * L // 128

ITCH = 128
NIT = BPW // ITCH

_mesh = plsc.VectorSubcoreMesh(
    core_axis_name="c", subcore_axis_name="s", num_cores=NC, num_subcores=NS
)


def _transpose_block(v2, vt):
    iota32 = lax.iota(jnp.int32, 16) * D

    @plsc.parallel_loop(0, BWW // 16, unroll=2)
    def group(g):
        w0 = g * 16
        gbase = w0 * D
        for f in range(D):
            vals = v2[f, pl.ds(w0, 16)]
            plsc.store_scatter(vt, [iota32 + (gbase + f)], vals)


def _convert_table(tabT_hbm, out_hbm, wid, v2a, v2b, vta, vtb,
                   isemA, isemB, osemA, osemB):
    def blkno(k):
        return jnp.minimum(k * NW + wid, NBLK - 1)

    def src_at(k):
        return tabT_hbm.at[:, pl.ds(blkno(k) * BWW, BWW)]

    def dst_at(k):
        return out_hbm.at[pl.ds(blkno(k) * (BWW * D), BWW * D)]

    pltpu.async_copy(src_at(0), v2a, isemA)
    pltpu.async_copy(src_at(1), v2b, isemB)

    def pair(q, carry):
        k = q * 2
        pltpu.make_async_copy(src_at(0), v2a, isemA).wait()
        _transpose_block(v2a, vta)
        outA = pltpu.async_copy(vta, dst_at(k), osemA)
        pltpu.async_copy(src_at_dyn_a(k + 2), v2a, isemA)
        pltpu.make_async_copy(src_at(1), v2b, isemB).wait()
        _transpose_block(v2b, vtb)
        outB = pltpu.async_copy(vtb, dst_at(k + 1), osemB)
        pltpu.async_copy(src_at_dyn_b(k + 3), v2b, isemB)
        outA.wait()
        outB.wait()
        return carry

    def src_at_dyn_a(k):
        return src_at(k)

    def src_at_dyn_b(k):
        return src_at(k)

    lax.fori_loop(0, NPAIR, pair, 0)
    pltpu.make_async_copy(src_at(0), v2a, isemA).wait()
    pltpu.make_async_copy(src_at(1), v2b, isemB).wait()


def _convert_tail(tail_hbm, out_hbm, v2t, vtt):
    pltpu.sync_copy(tail_hbm, v2t)

    def row(w, carry):
        vtt[pl.ds(w * D, 16)] = v2t[w, pl.ds(0, 16)]
        vtt[pl.ds(w * D + 16, 16)] = v2t[w, pl.ds(16, 16)]
        return carry

    lax.fori_loop(0, TAIL, row, 0)
    pltpu.sync_copy(vtt, out_hbm.at[pl.ds((V - TAIL) * D, TAIL * D)])


@functools.partial(
    pl.kernel,
    out_type=(
        jax.ShapeDtypeStruct((V * D,), jnp.float32),
        jax.ShapeDtypeStruct((B * L,), jnp.int32),
    ),
    mesh=_mesh,
    compiler_params=pltpu.CompilerParams(
        use_tc_tiling_on_sc=True, needs_layout_passes=False
    ),
    scratch_types=[
        pltpu.VMEM((32, BWW), jnp.float32),
        pltpu.VMEM((32, BWW), jnp.float32),
        pltpu.VMEM((BWW * D,), jnp.float32),
        pltpu.VMEM((BWW * D,), jnp.float32),
        pltpu.VMEM((TAIL, D), jnp.float32),
        pltpu.VMEM((TAIL * D,), jnp.float32),
        pltpu.VMEM((FCH, L), jnp.int32),
        pltpu.VMEM((FCH * L,), jnp.int32),
        pltpu.SemaphoreType.DMA,
        pltpu.SemaphoreType.DMA,
        pltpu.SemaphoreType.DMA,
        pltpu.SemaphoreType.DMA,
    ],
)
def _sc_format(
    wordT_hbm, wtail_hbm, qw_hbm,
    wout_hbm, qwf_hbm,
    v2a, v2b, vta, vtb, v2t, vtt, vf2, vf1,
    isemA, isemB, osemA, osemB,
):
    wid = lax.axis_index("s") * NC + lax.axis_index("c")
    _convert_table(wordT_hbm, wout_hbm, wid, v2a, v2b, vta, vtb,
                   isemA, isemB, osemA, osemB)
    _convert_tail(wtail_hbm, wout_hbm, v2t, vtt)

    base = wid * BPW

    def fchunk(c, carry):
        pltpu.sync_copy(qw_hbm.at[pl.ds(base + c * FCH, FCH)], vf2)

        def row(r, rc):
            o = r * L
            vf1[pl.ds(o, 16)] = vf2[r, pl.ds(0, 16)]
            vf1[pl.ds(o + 16, 16)] = vf2[r, pl.ds(16, 16)]
            vf1[pl.ds(o + 32, 16)] = vf2[r, pl.ds(32, 16)]
            vf1[pl.ds(o + 34, 16)] = vf2[r, pl.ds(34, 16)]
            return rc

        lax.fori_loop(0, FCH, row, 0)
        pltpu.sync_copy(vf1, qwf_hbm.at[pl.ds((base + c * FCH) * L, FCH * L)])
        return carry

    lax.fori_loop(0, NFCH, fchunk, 0)


@functools.partial(
    pl.kernel,
    out_type=(
        jax.ShapeDtypeStruct((B, D), jnp.float32),
        jax.ShapeDtypeStruct((B, D), jnp.float32),
    ),
    mesh=_mesh,
    compiler_params=pltpu.CompilerParams(use_tc_tiling_on_sc=False),
    scratch_types=[
        pltpu.VMEM((CH * L,), jnp.int32),
        pltpu.VMEM((CH * L, D), jnp.float32),
        pltpu.VMEM((CH, D), jnp.float32),
        pltpu.VMEM((BPW,), jnp.int32),
        pltpu.VMEM((ITCH, D), jnp.float32),
        pltpu.SemaphoreType.DMA,
    ],
)
def _sc_gather_pool(
    items_hbm, qwf_hbm, word_hbm, item_hbm,
    pooled_hbm, iout_hbm,
    idx_v, rows_v, pooled_v, iidx_v, irows_v, sem,
):
    wid = lax.axis_index("s") * NC + lax.axis_index("c")
    base = wid * BPW

    pltpu.sync_copy(items_hbm.at[pl.ds(base, BPW)], iidx_v)

    def chunk_body(c, carry):
        pltpu.sync_copy(
            qwf_hbm.at[pl.ds((base + c * CH) * L, CH * L)], idx_v
        )
        cps = [
            pltpu.async_copy(
                word_hbm.at[idx_v.at[pl.ds(j * 128, 128)]],
                rows_v.at[pl.ds(j * 128, 128)],
                sem,
            )
            for j in range(GPC)
        ]
        for cp in cps:
            cp.wait()

        def item_body(b, acc_carry):
            r0 = b * L
            acc0 = rows_v[r0, pl.ds(0, 16)]
            acc1 = rows_v[r0, pl.ds(16, 16)]
            for l in range(1, L):
                acc0 = acc0 + rows_v[r0 + l, pl.ds(0, 16)]
                acc1 = acc1 + rows_v[r0 + l, pl.ds(16, 16)]
            pooled_v[b, pl.ds(0, 16)] = acc0
            pooled_v[b, pl.ds(16, 16)] = acc1
            return acc_carry

        lax.fori_loop(0, CH, item_body, 0)
        pltpu.sync_copy(pooled_v, pooled_hbm.at[pl.ds(base + c * CH, CH)])
        return carry

    lax.fori_loop(0, NCHUNK, chunk_body, 0)

    for t in range(NIT):
        pltpu.async_copy(
            item_hbm.at[iidx_v.at[pl.ds(t * ITCH, ITCH)]], irows_v, sem
        ).wait()
        pltpu.sync_copy(irows_v, iout_hbm.at[pl.ds(base + t * ITCH, ITCH)])


_TB = 2048


def _tc_proj_body(x_ref, w_ref, b_ref, o_ref):
    x = x_ref[...] * np.float32(1.0 / L)
    y = lax.dot_general(
        x, w_ref[...], (((1,), (1,)), ((), ())),
        preferred_element_type=jnp.float32,
    )
    o_ref[...] = jnp.tanh(y + b_ref[...])


_tc_proj = pl.pallas_call(
    _tc_proj_body,
    out_shape=jax.ShapeDtypeStruct((B, D), jnp.float32),
    grid=(B // _TB,),
    in_specs=[
        pl.BlockSpec((_TB, D), lambda i: (i, 0)),
        pl.BlockSpec((D, D), lambda i: (0, 0)),
        pl.BlockSpec((1, D), lambda i: (0, 0)),
    ],
    out_specs=pl.BlockSpec((_TB, D), lambda i: (i, 0)),
)


def kernel(items, query_words, word_table, item_table, W_q, b_q):
    items = items.astype(jnp.int32)
    query_words = query_words.astype(jnp.int32)
    word_c, qw_flat = _sc_format(
        word_table.T, word_table[NBLK * BWW:], query_words,
    )
    word_2 = word_c.reshape(V, D)
    item_2 = item_table
    pooled_sum, item_emb = _sc_gather_pool(
        items, qw_flat, word_2, item_2
    )
    q = _tc_proj(pooled_sum, W_q, b_q.reshape(1, D))
    return (q, item_emb)
